# baseline (speedup 1.0000x reference)
import functools

import jax
import jax.numpy as jnp
from jax import lax
from jax.experimental import pallas as pl
from jax.experimental.pallas import tpu as pltpu
from jax.experimental.pallas import tpu_sc as plsc

B = 16384
D = 64
UF = 128
SF = 128

NC = 2
NS = 16
NW = NC * NS
ROWS_W = B // NW
NCH = 4
CH = ROWS_W // NCH

NW1 = NS
NCH1 = B // (NW1 * CH)

G = 8
RB = B // G


def _embed_gather_body(idx_hbm, tab_hbm, out_hbm, idx_v, rows_v, sem):
    wid = lax.axis_index("s")
    pltpu.sync_copy(idx_hbm.at[wid], idx_v)
    cps = []
    for j in range(NCH1):
        cps.append(pltpu.async_copy(tab_hbm.at[idx_v.at[j]], rows_v.at[j], sem))
    for c in cps:
        c.wait()
    pltpu.sync_copy(rows_v, out_hbm.at[pl.ds(wid * NCH1, NCH1)])


def _make_embed_gather(v_rows):
    return pl.kernel(
        _embed_gather_body,
        out_type=jax.ShapeDtypeStruct((NW1 * NCH1, CH, D), jnp.float32),
        mesh=plsc.VectorSubcoreMesh(core_axis_name="c", subcore_axis_name="s",
                                    num_cores=1),
        scratch_types=[
            pltpu.VMEM((NCH1, CH), jnp.int32),
            pltpu.VMEM((NCH1, CH, D), jnp.float32),
            pltpu.SemaphoreType.DMA,
        ],
        compiler_params=pltpu.CompilerParams(use_tc_tiling_on_sc=False),
    )


_sc_gather_u = _make_embed_gather(1000000)
_sc_gather_s = _make_embed_gather(100000)


def _sc_bias_body(uid_hbm, sid_hbm, bu_hbm, bs_hbm,
                  bu_out, bs_out,
                  idx_u, idx_s, bu_v, bs_v, sem):
    wid = lax.axis_index("s") * NC + lax.axis_index("c")
    pltpu.sync_copy(uid_hbm.at[wid], idx_u)
    pltpu.sync_copy(sid_hbm.at[wid], idx_s)
    copies = []
    for j in range(NCH):
        copies.append(pltpu.async_copy(bu_hbm.at[idx_u.at[j]], bu_v.at[j], sem))
        copies.append(pltpu.async_copy(bs_hbm.at[idx_s.at[j]], bs_v.at[j], sem))
    for c in copies:
        c.wait()
    base = wid * NCH
    pltpu.sync_copy(bu_v, bu_out.at[pl.ds(base, NCH)])
    pltpu.sync_copy(bs_v, bs_out.at[pl.ds(base, NCH)])


_sc_bias = pl.kernel(
    _sc_bias_body,
    out_type=(
        jax.ShapeDtypeStruct((NW * NCH, CH), jnp.float32),
        jax.ShapeDtypeStruct((NW * NCH, CH), jnp.float32),
    ),
    mesh=plsc.VectorSubcoreMesh(core_axis_name="c", subcore_axis_name="s"),
    scratch_types=[
        pltpu.VMEM((NCH, CH), jnp.int32),
        pltpu.VMEM((NCH, CH), jnp.int32),
        pltpu.VMEM((NCH, CH), jnp.float32),
        pltpu.VMEM((NCH, CH), jnp.float32),
        pltpu.SemaphoreType.DMA,
    ],
    compiler_params=pltpu.CompilerParams(use_tc_tiling_on_sc=False),
)


def _tc_combine_body(uf_ref, sf_ref, wu_ref, ws_ref, bwu_ref, bws_ref,
                     eu_ref, es_ref, bu_ref, bs_ref, out_ref):
    fu = jnp.dot(uf_ref[...], wu_ref[...], preferred_element_type=jnp.float32,
                 precision=lax.Precision.HIGHEST)
    fu = jnp.maximum(fu + bwu_ref[...], 0.0)
    fs = jnp.dot(sf_ref[...], ws_ref[...], preferred_element_type=jnp.float32,
                 precision=lax.Precision.HIGHEST)
    fs = jnp.maximum(fs + bws_ref[...], 0.0)
    u = eu_ref[...] + fu
    s = es_ref[...] + fs
    comb = jnp.sum(u * s, axis=1)
    out_ref[0, 0, :] = comb + bu_ref[0, 0, :] + bs_ref[0, 0, :]


_tc_combine = pl.pallas_call(
    _tc_combine_body,
    grid=(G,),
    in_specs=[
        pl.BlockSpec((RB, UF), lambda i: (i, 0)),
        pl.BlockSpec((RB, SF), lambda i: (i, 0)),
        pl.BlockSpec((UF, D), lambda i: (0, 0)),
        pl.BlockSpec((SF, D), lambda i: (0, 0)),
        pl.BlockSpec((1, D), lambda i: (0, 0)),
        pl.BlockSpec((1, D), lambda i: (0, 0)),
        pl.BlockSpec((RB, D), lambda i: (i, 0)),
        pl.BlockSpec((RB, D), lambda i: (i, 0)),
        pl.BlockSpec((1, 1, RB), lambda i: (i, 0, 0)),
        pl.BlockSpec((1, 1, RB), lambda i: (i, 0, 0)),
    ],
    out_specs=pl.BlockSpec((1, 1, RB), lambda i: (i, 0, 0)),
    out_shape=jax.ShapeDtypeStruct((G, 1, RB), jnp.float32),
)


def kernel(user_id, sku_id, user_features, sku_features, E_user, b_user,
           E_sku, b_sku, W_user, bW_user, W_sku, bW_sku):
    uid = user_id.reshape(B).astype(jnp.int32)
    sid = sku_id.reshape(B).astype(jnp.int32)
    eu4 = _sc_gather_u(uid.reshape(NW1, NCH1, CH), E_user)
    es4 = _sc_gather_s(sid.reshape(NW1, NCH1, CH), E_sku)
    bu4, bs4 = _sc_bias(uid.reshape(NW, NCH, CH), sid.reshape(NW, NCH, CH),
                        b_user.reshape(-1), b_sku.reshape(-1))
    eu = eu4.reshape(B, D)
    es = es4.reshape(B, D)
    bu3 = bu4.reshape(G, 1, RB)
    bs3 = bs4.reshape(G, 1, RB)
    out3 = _tc_combine(user_features, sku_features, W_user, W_sku,
                       bW_user.reshape(1, D), bW_sku.reshape(1, D),
                       eu, es, bu3, bs3)
    return out3.reshape(B)


# speedup vs baseline: 1.5705x; 1.5705x over previous
import functools

import jax
import jax.numpy as jnp
from jax import lax
from jax.experimental import pallas as pl
from jax.experimental.pallas import tpu as pltpu
from jax.experimental.pallas import tpu_sc as plsc

B = 16384
D = 64
UF = 128
SF = 128

NC = 2
NS = 16
NW = NC * NS
ROWS_W = B // NW
NCH = 4
CH = ROWS_W // NCH

GRP = 32

G = 8
RB = B // G


def _sc_embed_body(ids_hbm, eu_hbm, es_hbm,
                   eu_out, es_out,
                   ids_v, rows_v, sem):
    wid = lax.axis_index("s") * NC + lax.axis_index("c")
    pltpu.sync_copy(ids_hbm.at[wid], ids_v)
    for t in range(2):
        src = eu_hbm if t == 0 else es_hbm
        dst = eu_out if t == 0 else es_out

        def fire(g):
            b = g % 2
            sv = ids_v[t, pl.ds(g * 16, 16)]
            return [pltpu.async_copy(src.at[pl.ds(sv[l], 1)],
                                     rows_v.at[b].at[pl.ds(l, 1)], sem)
                    for l in range(16)]

        prev = fire(0)
        for g in range(1, GRP):
            cur = fire(g)
            for c in prev:
                c.wait()
            pltpu.sync_copy(rows_v.at[(g - 1) % 2],
                            dst.at[wid * GRP + (g - 1)])
            prev = cur
        for c in prev:
            c.wait()
        pltpu.sync_copy(rows_v.at[(GRP - 1) % 2], dst.at[wid * GRP + GRP - 1])


_sc_embed = pl.kernel(
    _sc_embed_body,
    out_type=(
        jax.ShapeDtypeStruct((NW * GRP, 16, D), jnp.float32),
        jax.ShapeDtypeStruct((NW * GRP, 16, D), jnp.float32),
    ),
    mesh=plsc.VectorSubcoreMesh(core_axis_name="c", subcore_axis_name="s"),
    scratch_types=[
        pltpu.VMEM((2, ROWS_W), jnp.int32),
        pltpu.VMEM((2, 16, D), jnp.float32),
        pltpu.SemaphoreType.DMA,
    ],
    compiler_params=pltpu.CompilerParams(use_tc_tiling_on_sc=True),
)


def _sc_bias_body(uid_hbm, sid_hbm, bu_hbm, bs_hbm,
                  bu_out, bs_out,
                  idx_u, idx_s, bu_v, bs_v, sem):
    wid = lax.axis_index("s") * NC + lax.axis_index("c")
    pltpu.sync_copy(uid_hbm.at[wid], idx_u)
    pltpu.sync_copy(sid_hbm.at[wid], idx_s)
    copies = []
    for j in range(NCH):
        copies.append(pltpu.async_copy(bu_hbm.at[idx_u.at[j]], bu_v.at[j], sem))
        copies.append(pltpu.async_copy(bs_hbm.at[idx_s.at[j]], bs_v.at[j], sem))
    for c in copies:
        c.wait()
    base = wid * NCH
    pltpu.sync_copy(bu_v, bu_out.at[pl.ds(base, NCH)])
    pltpu.sync_copy(bs_v, bs_out.at[pl.ds(base, NCH)])


_sc_bias = pl.kernel(
    _sc_bias_body,
    out_type=(
        jax.ShapeDtypeStruct((NW * NCH, CH), jnp.float32),
        jax.ShapeDtypeStruct((NW * NCH, CH), jnp.float32),
    ),
    mesh=plsc.VectorSubcoreMesh(core_axis_name="c", subcore_axis_name="s"),
    scratch_types=[
        pltpu.VMEM((NCH, CH), jnp.int32),
        pltpu.VMEM((NCH, CH), jnp.int32),
        pltpu.VMEM((NCH, CH), jnp.float32),
        pltpu.VMEM((NCH, CH), jnp.float32),
        pltpu.SemaphoreType.DMA,
    ],
    compiler_params=pltpu.CompilerParams(use_tc_tiling_on_sc=False),
)


def _tc_combine_body(uf_ref, sf_ref, wu_ref, ws_ref, bwu_ref, bws_ref,
                     eu_ref, es_ref, bu_ref, bs_ref, out_ref):
    fu = jnp.dot(uf_ref[...], wu_ref[...], preferred_element_type=jnp.float32,
                 precision=lax.Precision.HIGHEST)
    fu = jnp.maximum(fu + bwu_ref[...], 0.0)
    fs = jnp.dot(sf_ref[...], ws_ref[...], preferred_element_type=jnp.float32,
                 precision=lax.Precision.HIGHEST)
    fs = jnp.maximum(fs + bws_ref[...], 0.0)
    u = eu_ref[...] + fu
    s = es_ref[...] + fs
    comb = jnp.sum(u * s, axis=1)
    out_ref[0, 0, :] = comb + bu_ref[0, 0, :] + bs_ref[0, 0, :]


_tc_combine = pl.pallas_call(
    _tc_combine_body,
    grid=(G,),
    in_specs=[
        pl.BlockSpec((RB, UF), lambda i: (i, 0)),
        pl.BlockSpec((RB, SF), lambda i: (i, 0)),
        pl.BlockSpec((UF, D), lambda i: (0, 0)),
        pl.BlockSpec((SF, D), lambda i: (0, 0)),
        pl.BlockSpec((1, D), lambda i: (0, 0)),
        pl.BlockSpec((1, D), lambda i: (0, 0)),
        pl.BlockSpec((RB, D), lambda i: (i, 0)),
        pl.BlockSpec((RB, D), lambda i: (i, 0)),
        pl.BlockSpec((1, 1, RB), lambda i: (i, 0, 0)),
        pl.BlockSpec((1, 1, RB), lambda i: (i, 0, 0)),
    ],
    out_specs=pl.BlockSpec((1, 1, RB), lambda i: (i, 0, 0)),
    out_shape=jax.ShapeDtypeStruct((G, 1, RB), jnp.float32),
)


def kernel(user_id, sku_id, user_features, sku_features, E_user, b_user,
           E_sku, b_sku, W_user, bW_user, W_sku, bW_sku):
    uid = user_id.reshape(B).astype(jnp.int32)
    sid = sku_id.reshape(B).astype(jnp.int32)
    ids = jnp.stack([uid.reshape(NW, ROWS_W), sid.reshape(NW, ROWS_W)], axis=1)
    eu4, es4 = _sc_embed(ids, E_user, E_sku)
    bu4, bs4 = _sc_bias(uid.reshape(NW, NCH, CH), sid.reshape(NW, NCH, CH),
                        b_user.reshape(-1), b_sku.reshape(-1))
    eu = eu4.reshape(B, D)
    es = es4.reshape(B, D)
    bu3 = bu4.reshape(G, 1, RB)
    bs3 = bs4.reshape(G, 1, RB)
    out3 = _tc_combine(user_features, sku_features, W_user, W_sku,
                       bW_user.reshape(1, D), bW_sku.reshape(1, D),
                       eu, es, bu3, bs3)
    return out3.reshape(B)


# speedup vs baseline: 1.5751x; 1.0029x over previous
import functools

import jax
import jax.numpy as jnp
from jax import lax
from jax.experimental import pallas as pl
from jax.experimental.pallas import tpu as pltpu
from jax.experimental.pallas import tpu_sc as plsc

B = 16384
D = 64
UF = 128
SF = 128

NC = 2
NS = 16
NW = NC * NS
ROWS_W = B // NW
NCH = 4
CH = ROWS_W // NCH

GRP = 32

G = 8
RB = B // G


def _sc_embed_body(ids_hbm, eu_hbm, es_hbm,
                   eu_out, es_out,
                   ids_v, rows_v, sem):
    wid = lax.axis_index("s") * NC + lax.axis_index("c")
    pltpu.sync_copy(ids_hbm.at[wid], ids_v)
    for t in range(2):
        src = eu_hbm if t == 0 else es_hbm
        dst = eu_out if t == 0 else es_out

        def fire(g):
            b = g % 2
            sv = ids_v[t, pl.ds(g * 16, 16)]
            return [pltpu.async_copy(src.at[pl.ds(sv[l], 1)],
                                     rows_v.at[b].at[pl.ds(l, 1)], sem)
                    for l in range(16)]

        prev = fire(0)
        for g in range(1, GRP):
            cur = fire(g)
            for c in prev:
                c.wait()
            pltpu.sync_copy(rows_v.at[(g - 1) % 2],
                            dst.at[wid * GRP + (g - 1)])
            prev = cur
        for c in prev:
            c.wait()
        pltpu.sync_copy(rows_v.at[(GRP - 1) % 2], dst.at[wid * GRP + GRP - 1])


_sc_embed = pl.kernel(
    _sc_embed_body,
    out_type=(
        jax.ShapeDtypeStruct((NW * GRP, 16, D), jnp.float32),
        jax.ShapeDtypeStruct((NW * GRP, 16, D), jnp.float32),
    ),
    mesh=plsc.VectorSubcoreMesh(core_axis_name="c", subcore_axis_name="s"),
    scratch_types=[
        pltpu.VMEM((2, ROWS_W), jnp.int32),
        pltpu.VMEM((2, 16, D), jnp.float32),
        pltpu.SemaphoreType.DMA,
    ],
    compiler_params=pltpu.CompilerParams(use_tc_tiling_on_sc=True),
)


def _sc_bias_body(uid_hbm, sid_hbm, bu_hbm, bs_hbm,
                  bu_out, bs_out,
                  idx_u, idx_s, bu_v, bs_v, sem):
    wid = lax.axis_index("s") * NC + lax.axis_index("c")
    pltpu.sync_copy(uid_hbm.at[wid], idx_u)
    pltpu.sync_copy(sid_hbm.at[wid], idx_s)
    copies = []
    for j in range(NCH):
        copies.append(pltpu.async_copy(bu_hbm.at[0].at[idx_u.at[j]],
                                       bu_v.at[j], sem))
        copies.append(pltpu.async_copy(bs_hbm.at[0].at[idx_s.at[j]],
                                       bs_v.at[j], sem))
    for c in copies:
        c.wait()
    base = wid * NCH
    pltpu.sync_copy(bu_v, bu_out.at[pl.ds(base, NCH)])
    pltpu.sync_copy(bs_v, bs_out.at[pl.ds(base, NCH)])


_sc_bias = pl.kernel(
    _sc_bias_body,
    out_type=(
        jax.ShapeDtypeStruct((NW * NCH, CH), jnp.float32),
        jax.ShapeDtypeStruct((NW * NCH, CH), jnp.float32),
    ),
    mesh=plsc.VectorSubcoreMesh(core_axis_name="c", subcore_axis_name="s"),
    scratch_types=[
        pltpu.VMEM((NCH, CH), jnp.int32),
        pltpu.VMEM((NCH, CH), jnp.int32),
        pltpu.VMEM((NCH, CH), jnp.float32),
        pltpu.VMEM((NCH, CH), jnp.float32),
        pltpu.SemaphoreType.DMA,
    ],
    compiler_params=pltpu.CompilerParams(use_tc_tiling_on_sc=False),
)


def _tc_combine_body(uf_ref, sf_ref, wu_ref, ws_ref, bwu_ref, bws_ref,
                     eu_ref, es_ref, bu_ref, bs_ref, out_ref):
    fu = jnp.dot(uf_ref[...], wu_ref[...], preferred_element_type=jnp.float32,
                 precision=lax.Precision.HIGHEST)
    fu = jnp.maximum(fu + bwu_ref[...], 0.0)
    fs = jnp.dot(sf_ref[...], ws_ref[...], preferred_element_type=jnp.float32,
                 precision=lax.Precision.HIGHEST)
    fs = jnp.maximum(fs + bws_ref[...], 0.0)
    u = eu_ref[...] + fu
    s = es_ref[...] + fs
    comb = jnp.sum(u * s, axis=1)
    out_ref[0, 0, :] = comb + bu_ref[0, 0, :] + bs_ref[0, 0, :]


_tc_combine = pl.pallas_call(
    _tc_combine_body,
    grid=(G,),
    in_specs=[
        pl.BlockSpec((RB, UF), lambda i: (i, 0)),
        pl.BlockSpec((RB, SF), lambda i: (i, 0)),
        pl.BlockSpec((UF, D), lambda i: (0, 0)),
        pl.BlockSpec((SF, D), lambda i: (0, 0)),
        pl.BlockSpec((1, D), lambda i: (0, 0)),
        pl.BlockSpec((1, D), lambda i: (0, 0)),
        pl.BlockSpec((RB, D), lambda i: (i, 0)),
        pl.BlockSpec((RB, D), lambda i: (i, 0)),
        pl.BlockSpec((1, 1, RB), lambda i: (i, 0, 0)),
        pl.BlockSpec((1, 1, RB), lambda i: (i, 0, 0)),
    ],
    out_specs=pl.BlockSpec((1, 1, RB), lambda i: (i, 0, 0)),
    out_shape=jax.ShapeDtypeStruct((G, 1, RB), jnp.float32),
)


def kernel(user_id, sku_id, user_features, sku_features, E_user, b_user,
           E_sku, b_sku, W_user, bW_user, W_sku, bW_sku):
    uid = user_id.reshape(B).astype(jnp.int32)
    sid = sku_id.reshape(B).astype(jnp.int32)
    ids = jnp.stack([uid.reshape(NW, ROWS_W), sid.reshape(NW, ROWS_W)], axis=1)
    eu4, es4 = _sc_embed(ids, E_user, E_sku)
    bu4, bs4 = _sc_bias(uid.reshape(NW, NCH, CH), sid.reshape(NW, NCH, CH),
                        b_user.T, b_sku.T)
    eu = eu4.reshape(B, D)
    es = es4.reshape(B, D)
    bu3 = bu4.reshape(G, 1, RB)
    bs3 = bs4.reshape(G, 1, RB)
    out3 = _tc_combine(user_features, sku_features, W_user, W_sku,
                       bW_user.reshape(1, D), bW_sku.reshape(1, D),
                       eu, es, bu3, bs3)
    return out3.reshape(B)
